# baseline (speedup 1.0000x reference)
import functools

import jax
import jax.numpy as jnp
from jax import lax
from jax.experimental import pallas as pl
from jax.experimental.pallas import tpu as pltpu
from jax.experimental.pallas import tpu_sc as plsc

_NC = 2
_NS = 16
_NW = _NC * _NS

_CHUNK = 32
_NBUF = 3


@functools.cache
def _make_gather(B, S, V, D):
    N = B * S
    n_per_w = N // _NW
    n_chunks = n_per_w // _CHUNK
    w_per_row = S // n_per_w
    mesh = plsc.VectorSubcoreMesh(core_axis_name="c", subcore_axis_name="s")

    rows_t = [pltpu.VMEM((_CHUNK, D), jnp.float32) for _ in range(_NBUF)]
    gsem_t = [pltpu.SemaphoreType.DMA for _ in range(_NBUF)]
    wsem_t = [pltpu.SemaphoreType.DMA for _ in range(_NBUF)]

    @functools.partial(
        pl.kernel,
        out_type=jax.ShapeDtypeStruct((N, D), jnp.float32),
        mesh=mesh,
        scratch_types=[pltpu.VMEM((n_per_w,), jnp.int32)]
        + rows_t + gsem_t + wsem_t,
    )
    def k(table_hbm, idx_hbm, out_hbm, idx_v, *bufs):
        rows = bufs[:_NBUF]
        gsem = bufs[_NBUF:2 * _NBUF]
        wsem = bufs[2 * _NBUF:]
        wid = lax.axis_index("s") * _NC + lax.axis_index("c")
        base = wid * n_per_w
        pltpu.sync_copy(
            idx_hbm.at[wid // w_per_row,
                       pl.ds((wid % w_per_row) * n_per_w, n_per_w)],
            idx_v)

        def gather(c):
            b = c % _NBUF
            return pltpu.async_copy(
                table_hbm.at[idx_v.at[pl.ds(c * _CHUNK, _CHUNK)]],
                rows[b], gsem[b])

        def write(c):
            b = c % _NBUF
            return pltpu.async_copy(
                rows[b], out_hbm.at[pl.ds(base + c * _CHUNK, _CHUNK), :],
                wsem[b])

        g = [None] * n_chunks
        w = [None] * n_chunks
        for c in range(min(_NBUF, n_chunks)):
            g[c] = gather(c)
        for c in range(n_chunks):
            g[c].wait()
            w[c] = write(c)
            if c + _NBUF < n_chunks:
                w[c].wait()
                g[c + _NBUF] = gather(c + _NBUF)
        for c in range(max(0, n_chunks - _NBUF), n_chunks):
            w[c].wait()

    return k


def kernel(inputs, shared_weights):
    B, S = inputs.shape
    V, D = shared_weights.shape
    out = _make_gather(B, S, V, D)(shared_weights, inputs)
    return out.reshape(B, S, D)


# speedup vs baseline: 1.0144x; 1.0144x over previous
import functools

import jax
import jax.numpy as jnp
from jax import lax
from jax.experimental import pallas as pl
from jax.experimental.pallas import tpu as pltpu
from jax.experimental.pallas import tpu_sc as plsc

_NC = 2
_NS = 16
_NW = _NC * _NS

_CHUNK = 16
_NBUF = 7


@functools.cache
def _make_gather(B, S, V, D):
    N = B * S
    n_per_w = N // _NW
    n_chunks = n_per_w // _CHUNK
    w_per_row = S // n_per_w
    mesh = plsc.VectorSubcoreMesh(core_axis_name="c", subcore_axis_name="s")

    rows_t = [pltpu.VMEM((_CHUNK, D), jnp.float32) for _ in range(_NBUF)]
    gsem_t = [pltpu.SemaphoreType.DMA for _ in range(_NBUF)]
    wsem_t = [pltpu.SemaphoreType.DMA for _ in range(_NBUF)]

    @functools.partial(
        pl.kernel,
        out_type=jax.ShapeDtypeStruct((N, D), jnp.float32),
        mesh=mesh,
        scratch_types=[pltpu.VMEM((n_per_w,), jnp.int32)]
        + rows_t + gsem_t + wsem_t,
    )
    def k(table_hbm, idx_hbm, out_hbm, idx_v, *bufs):
        rows = bufs[:_NBUF]
        gsem = bufs[_NBUF:2 * _NBUF]
        wsem = bufs[2 * _NBUF:]
        wid = lax.axis_index("s") * _NC + lax.axis_index("c")
        base = wid * n_per_w
        pltpu.sync_copy(
            idx_hbm.at[wid // w_per_row,
                       pl.ds((wid % w_per_row) * n_per_w, n_per_w)],
            idx_v)

        def gather(c):
            b = c % _NBUF
            return pltpu.async_copy(
                table_hbm.at[idx_v.at[pl.ds(c * _CHUNK, _CHUNK)]],
                rows[b], gsem[b])

        def write(c):
            b = c % _NBUF
            return pltpu.async_copy(
                rows[b], out_hbm.at[pl.ds(base + c * _CHUNK, _CHUNK), :],
                wsem[b])

        g = [None] * n_chunks
        w = [None] * n_chunks
        for c in range(min(_NBUF, n_chunks)):
            g[c] = gather(c)
        for c in range(n_chunks):
            g[c].wait()
            w[c] = write(c)
            if c + _NBUF < n_chunks:
                w[c].wait()
                g[c + _NBUF] = gather(c + _NBUF)
        for c in range(max(0, n_chunks - _NBUF), n_chunks):
            w[c].wait()

    return k


def kernel(inputs, shared_weights):
    B, S = inputs.shape
    V, D = shared_weights.shape
    out = _make_gather(B, S, V, D)(shared_weights, inputs)
    return out.reshape(B, S, D)
